# baseline (speedup 1.0000x reference)
import functools

import jax
import jax.numpy as jnp
from jax import lax
from jax.experimental import pallas as pl
from jax.experimental.pallas import tpu as pltpu
from jax.experimental.pallas import tpu_sc as plsc

_NC = 2
_NS = 16
_NW = _NC * _NS
_LANES = 16

_RND = 12582912.0

_GPC = 32


def _make_sc_lookup(n, nvoc, D):
    per = n // _NW
    ngrp = per // _LANES
    npair = ngrp // (2 * _GPC)
    chw = _GPC * _LANES * D
    mesh = plsc.VectorSubcoreMesh(core_axis_name="c", subcore_axis_name="s")

    @functools.partial(
        pl.kernel,
        mesh=mesh,
        out_type=jax.ShapeDtypeStruct((n * D,), jnp.float32),
        scratch_types=[
            pltpu.VMEM((per,), jnp.float32),
            pltpu.VMEM((nvoc * D,), jnp.float32),
            pltpu.VMEM((chw,), jnp.float32),
            pltpu.VMEM((chw,), jnp.float32),
            pltpu.SemaphoreType.DMA,
            pltpu.SemaphoreType.DMA,
        ],
        compiler_params=pltpu.CompilerParams(
            use_tc_tiling_on_sc=False, needs_layout_passes=False
        ),
    )
    def run(x_hbm, tab_hbm, out_hbm, x_v, tab_v, buf0, buf1, wsem0, wsem1):
        wid = lax.axis_index("s") * _NC + lax.axis_index("c")
        ebase = wid * per
        obase = ebase * D
        pltpu.sync_copy(x_hbm.at[pl.ds(ebase, per)], x_v)
        pltpu.sync_copy(tab_hbm, tab_v)

        ramp = lax.iota(jnp.int32, _LANES) * D

        def expand(buf, c):
            def grp(g, carry):
                xv = x_v[pl.ds((c * _GPC + g) * _LANES, _LANES)]
                v = jnp.minimum(jnp.maximum(xv, -65.0), 65.0)
                rr = (v + _RND) - _RND
                t = rr.astype(jnp.int32)
                t = jnp.minimum(jnp.maximum(t, -64), 64) + 65
                t = jnp.where(xv == jnp.inf, 130, t)
                t = jnp.where(xv == -jnp.inf, 0, t)
                t = jnp.where(xv != xv, 0, t)
                src = t * D
                dst = g * (_LANES * D) + ramp
                for d in range(D):
                    plsc.store_scatter(
                        buf, [dst + d], plsc.load_gather(tab_v, [src + d])
                    )
                return carry

            lax.fori_loop(0, _GPC, grp, 0)

        def body(i, carry):
            for b, buf, wsem in ((0, buf0, wsem0), (1, buf1, wsem1)):
                c = 2 * i + b

                @pl.when(i >= 1)
                def _wait():
                    pltpu.make_async_copy(
                        buf, out_hbm.at[pl.ds(0, chw)], wsem
                    ).wait()

                expand(buf, c)
                pltpu.async_copy(
                    buf, out_hbm.at[pl.ds(obase + c * chw, chw)], wsem
                )
            return carry

        lax.fori_loop(0, npair, body, 0)
        pltpu.make_async_copy(buf0, out_hbm.at[pl.ds(0, chw)], wsem0).wait()
        pltpu.make_async_copy(buf1, out_hbm.at[pl.ds(0, chw)], wsem1).wait()

    return run


def kernel(x, table):
    b, seq = x.shape[0], x.shape[1]
    D = table.shape[1]
    n = b * seq
    out = _make_sc_lookup(n, table.shape[0], D)(
        x.reshape(n), table.reshape(-1)
    )
    return out.reshape(b, seq, D)


# speedup vs baseline: 1.9140x; 1.9140x over previous
import functools

import jax
import jax.numpy as jnp
from jax import lax
from jax.experimental import pallas as pl
from jax.experimental.pallas import tpu as pltpu
from jax.experimental.pallas import tpu_sc as plsc

_NC = 2
_NS = 16
_NW = _NC * _NS
_LANES = 16

_RND = 12582912.0

_CH = 256
_NB = 4


def _make_sc_lookup(n, D):
    per = n // _NW
    nch = per // _CH
    mesh = plsc.VectorSubcoreMesh(core_axis_name="c", subcore_axis_name="s")

    @functools.partial(
        pl.kernel,
        mesh=mesh,
        out_type=jax.ShapeDtypeStruct((n, D), jnp.float32),
        scratch_types=[
            pltpu.VMEM((per,), jnp.float32),
            pltpu.VMEM((per,), jnp.int32),
            pltpu.VMEM((_NB, _CH, D), jnp.float32),
        ]
        + [pltpu.SemaphoreType.DMA] * (2 * _NB),
        compiler_params=pltpu.CompilerParams(use_tc_tiling_on_sc=False),
    )
    def run(x_hbm, tab_hbm, out_hbm, x_v, idx_v, buf, *sems):
        gsem, wsem = sems[:_NB], sems[_NB:]
        wid = lax.axis_index("s") * _NC + lax.axis_index("c")
        row0 = wid * per
        pltpu.sync_copy(x_hbm.at[pl.ds(row0, per)], x_v)

        def grp(g, carry):
            xv = x_v[pl.ds(g * _LANES, _LANES)]
            v = jnp.minimum(jnp.maximum(xv, -65.0), 65.0)
            rr = (v + _RND) - _RND
            t = rr.astype(jnp.int32)
            t = jnp.minimum(jnp.maximum(t, -64), 64) + 65
            t = jnp.where(xv == jnp.inf, 130, t)
            t = jnp.where(xv == -jnp.inf, 0, t)
            t = jnp.where(xv != xv, 0, t)
            idx_v[pl.ds(g * _LANES, _LANES)] = t
            return carry

        lax.fori_loop(0, per // _LANES, grp, 0)

        def gather(c):
            b = c % _NB
            return [
                pltpu.async_copy(
                    tab_hbm.at[idx_v.at[pl.ds(c * _CH + j * 128, 128)]],
                    buf.at[b, pl.ds(j * 128, 128)],
                    gsem[b],
                )
                for j in range(_CH // 128)
            ]

        gathers = {c: gather(c) for c in range(2)}
        writes = {}
        for c in range(nch):
            b = c % _NB
            if c + 2 < nch:
                if c - 2 >= 0:
                    writes.pop(c - 2).wait()
                gathers[c + 2] = gather(c + 2)
            for cp in gathers.pop(c):
                cp.wait()
            writes[c] = pltpu.async_copy(
                buf.at[b], out_hbm.at[pl.ds(row0 + c * _CH, _CH)], wsem[b]
            )
        for c in sorted(writes):
            writes.pop(c).wait()

    return run


def kernel(x, table):
    b, seq = x.shape[0], x.shape[1]
    D = table.shape[1]
    n = b * seq
    out = _make_sc_lookup(n, D)(x.reshape(n), table)
    return out.reshape(b, seq, D)


# speedup vs baseline: 3.2356x; 1.6905x over previous
import functools

import jax
import jax.numpy as jnp
from jax import lax
from jax.experimental import pallas as pl
from jax.experimental.pallas import tpu as pltpu
from jax.experimental.pallas import tpu_sc as plsc

_NC = 2
_NS = 16
_NW = _NC * _NS
_LANES = 16

_RND = 12582912.0

_CH = 256
_NB = 4


def _make_sc_lookup(n, nvoc, D):
    per = n // _NW
    nch = per // _CH
    mesh = plsc.VectorSubcoreMesh(core_axis_name="c", subcore_axis_name="s")

    @functools.partial(
        pl.kernel,
        mesh=mesh,
        out_type=jax.ShapeDtypeStruct((n, D), jnp.float32),
        scratch_types=[
            pltpu.VMEM((per,), jnp.float32),
            pltpu.VMEM((per,), jnp.int32),
            pltpu.VMEM((_NB, _CH, D), jnp.float32),
        ]
        + [pltpu.SemaphoreType.DMA] * (2 * _NB),
        compiler_params=pltpu.CompilerParams(use_tc_tiling_on_sc=False),
    )
    def run(x_hbm, tab_hbm, out_hbm, x_v, idx_v, buf, *sems):
        gsem, wsem = sems[:_NB], sems[_NB:]
        wid = lax.axis_index("s") * _NC + lax.axis_index("c")
        row0 = wid * per
        tab_off = wid * nvoc
        pltpu.sync_copy(x_hbm.at[pl.ds(row0, per)], x_v)

        def grp(g, carry):
            xv = x_v[pl.ds(g * _LANES, _LANES)]
            v = jnp.minimum(jnp.maximum(xv, -65.0), 65.0)
            rr = (v + _RND) - _RND
            t = rr.astype(jnp.int32)
            t = jnp.minimum(jnp.maximum(t, -64), 64) + 65
            t = jnp.where(xv == jnp.inf, 130, t)
            t = jnp.where(xv == -jnp.inf, 0, t)
            t = jnp.where(xv != xv, 0, t)
            idx_v[pl.ds(g * _LANES, _LANES)] = t + tab_off
            return carry

        lax.fori_loop(0, per // _LANES, grp, 0)

        def gather(c):
            b = c % _NB
            return [
                pltpu.async_copy(
                    tab_hbm.at[idx_v.at[pl.ds(c * _CH + j * 128, 128)]],
                    buf.at[b, pl.ds(j * 128, 128)],
                    gsem[b],
                )
                for j in range(_CH // 128)
            ]

        gathers = {c: gather(c) for c in range(2)}
        writes = {}
        for c in range(nch):
            b = c % _NB
            if c + 2 < nch:
                if c - 2 >= 0:
                    writes.pop(c - 2).wait()
                gathers[c + 2] = gather(c + 2)
            for cp in gathers.pop(c):
                cp.wait()
            writes[c] = pltpu.async_copy(
                buf.at[b], out_hbm.at[pl.ds(row0 + c * _CH, _CH)], wsem[b]
            )
        for c in sorted(writes):
            writes.pop(c).wait()

    return run


def kernel(x, table):
    b, seq = x.shape[0], x.shape[1]
    D = table.shape[1]
    n = b * seq
    out = _make_sc_lookup(n, table.shape[0], D)(
        x.reshape(n), jnp.tile(table, (_NW, 1))
    )
    return out.reshape(b, seq, D)
